# Initial kernel scaffold; baseline (speedup 1.0000x reference)
import functools

import jax
import jax.numpy as jnp
from jax import lax
from jax.experimental import pallas as pl
from jax.experimental.pallas import tpu as pltpu
from jax.experimental.pallas import tpu_sc as plsc

N = 10000
E = 320000
DIN = 128
DOUT = 32
KH = 4
DH = KH * DOUT

NC = 2
NS = 16
NW = NC * NS
T = E // NW
CH = 128
TP = ((T + CH - 1) // CH) * CH
NG = TP // 16
NPH = 10240
TAB = KH * NPH
SLICE = TAB // NS
ROWS_PER_TILE = N // NS

_mesh = functools.partial(
    plsc.VectorSubcoreMesh, core_axis_name="c", subcore_axis_name="s",
    num_cores=NC, num_subcores=NS)


def _wid():
  return lax.axis_index("s") * NC + lax.axis_index("c")


def _lanes():
  return lax.broadcasted_iota(jnp.int32, (16,), 0)


def _tc1_body(x_ref, wt_ref, am_ref, h_ref, st_ref):
  x = x_ref[...]
  h = jnp.dot(x, wt_ref[...], preferred_element_type=jnp.float32)
  h_ref[...] = h
  st_ref[...] = lax.dot_general(
      am_ref[...], h, (((0,), (1,)), ((), ())),
      preferred_element_type=jnp.float32)


def _tc1(X, Wt, Amat):
  blk = 500
  grid = N // blk
  return pl.pallas_call(
      _tc1_body,
      grid=(grid,),
      in_specs=[
          pl.BlockSpec((blk, DIN), lambda i: (i, 0)),
          pl.BlockSpec((DIN, DH), lambda i: (0, 0)),
          pl.BlockSpec((DIN, 2 * KH), lambda i: (0, 0)),
      ],
      out_specs=[
          pl.BlockSpec((blk, DH), lambda i: (i, 0)),
          pl.BlockSpec((2 * KH, blk), lambda i: (0, i)),
      ],
      out_shape=[
          jax.ShapeDtypeStruct((N, DH), jnp.float32),
          jax.ShapeDtypeStruct((2 * KH, N), jnp.float32),
      ],
  )(X, Wt, Amat)


def _ka_body(sst, dst_h, src_h, scores_h, m2_h,
             ss_v, sn_v, m_v, dst_v, src_v, sc_v, shr, acc_v, tmp_v):
  wid = _wid()
  sid = lax.axis_index("s")
  core = lax.axis_index("c")
  pltpu.sync_copy(dst_h.at[wid], dst_v)
  pltpu.sync_copy(src_h.at[wid], src_v)

  def init(i, _):
    m_v[pl.ds(pl.multiple_of(i * 16, 16), 16)] = jnp.full((16,), -1e30,
                                                          jnp.float32)
    return 0
  lax.fori_loop(0, TAB // 16, init, 0)

  lanes = _lanes()
  for k in range(KH):
    pltpu.sync_copy(sst.at[k], ss_v)
    pltpu.sync_copy(sst.at[KH + k], sn_v)

    def grp(g, _, k=k):
      off = pl.multiple_of(g * 16, 16)
      d = dst_v[pl.ds(off, 16)]
      s = src_v[pl.ds(off, 16)]
      mask = (g * 16 + lanes) < T
      sc = plsc.load_gather(ss_v, [d]) + plsc.load_gather(sn_v, [s])
      sc = jnp.where(sc > 0, sc, sc * jnp.float32(0.01))
      sc_v[pl.ds(off, 16)] = sc
      idx = d + k * NPH

      def cond(act):
        return jnp.any(act)

      def body(act):
        cur = plsc.load_gather(m_v, [idx])
        need = (sc > cur) & act
        plsc.store_scatter(m_v, [idx], sc, mask=need)
        cur2 = plsc.load_gather(m_v, [idx])
        return (sc > cur2) & act

      lax.while_loop(cond, body, mask)
      return 0

    lax.fori_loop(0, NG, grp, 0)
    pltpu.sync_copy(sc_v, scores_h.at[k, wid])

  pltpu.sync_copy(m_v, shr.at[sid])
  plsc.subcore_barrier()
  base = pl.multiple_of(sid * SLICE, 8)
  pltpu.sync_copy(shr.at[0, pl.ds(base, SLICE)], acc_v)
  for t in range(1, NS):
    pltpu.sync_copy(shr.at[t, pl.ds(base, SLICE)], tmp_v)

    def mx(i, _):
      o = pl.multiple_of(i * 16, 16)
      acc_v[pl.ds(o, 16)] = jnp.maximum(acc_v[pl.ds(o, 16)],
                                        tmp_v[pl.ds(o, 16)])
      return 0
    lax.fori_loop(0, SLICE // 16, mx, 0)
  pltpu.sync_copy(acc_v, m2_h.at[core, pl.ds(base, SLICE)])


def _ka(SSt, dstp, srcp):
  return pl.kernel(
      _ka_body,
      out_type=[
          jax.ShapeDtypeStruct((KH, NW, TP), jnp.float32),
          jax.ShapeDtypeStruct((NC, TAB), jnp.float32),
      ],
      mesh=_mesh(),
      scratch_types=[
          pltpu.VMEM((N,), jnp.float32),
          pltpu.VMEM((N,), jnp.float32),
          pltpu.VMEM((TAB,), jnp.float32),
          pltpu.VMEM((TP,), jnp.int32),
          pltpu.VMEM((TP,), jnp.int32),
          pltpu.VMEM((TP,), jnp.float32),
          pltpu.VMEM_SHARED((NS, TAB), jnp.float32),
          pltpu.VMEM((SLICE,), jnp.float32),
          pltpu.VMEM((SLICE,), jnp.float32),
      ],
  )(SSt, dstp, srcp)


def _kb_body(scores_h, m2_h, dst_h, ex_h, den2_h,
             m_v, den_v, dst_v, sc_v, ex_v, claim_v, t0_v, t1_v, shr,
             acc_v, tmp_v):
  wid = _wid()
  sid = lax.axis_index("s")
  core = lax.axis_index("c")
  pltpu.sync_copy(dst_h.at[wid], dst_v)

  CKW = 2048
  for ci in range(TAB // CKW):
    pltpu.sync_copy(m2_h.at[0, pl.ds(ci * CKW, CKW)], t0_v)
    pltpu.sync_copy(m2_h.at[1, pl.ds(ci * CKW, CKW)], t1_v)

    def mrg(i, _, ci=ci):
      o = pl.multiple_of(i * 16, 16)
      m_v[pl.ds(pl.multiple_of(ci * CKW, 16) + o, 16)] = jnp.maximum(
          t0_v[pl.ds(o, 16)], t1_v[pl.ds(o, 16)])
      return 0
    lax.fori_loop(0, CKW // 16, mrg, 0)

  def zero(i, _):
    den_v[pl.ds(pl.multiple_of(i * 16, 16), 16)] = jnp.zeros((16,),
                                                             jnp.float32)
    return 0
  lax.fori_loop(0, TAB // 16, zero, 0)

  lanes = _lanes()
  for k in range(KH):
    pltpu.sync_copy(scores_h.at[k, wid], sc_v)

    def grp(g, _, k=k):
      off = pl.multiple_of(g * 16, 16)
      d = dst_v[pl.ds(off, 16)]
      sc = sc_v[pl.ds(off, 16)]
      mask = (g * 16 + lanes) < T
      idx = d + k * NPH
      mx = plsc.load_gather(m_v, [idx])
      ex = jnp.where(mask, jnp.exp(sc - mx), jnp.float32(0.0))
      ex_v[pl.ds(off, 16)] = ex
      slot = lax.bitwise_and(idx, jnp.int32(63))

      def cond(act):
        return jnp.any(act)

      def body(act):
        plsc.store_scatter(claim_v, [slot], lanes, mask=act)
        win = (plsc.load_gather(claim_v, [slot]) == lanes) & act
        cur = plsc.load_gather(den_v, [idx])
        plsc.store_scatter(den_v, [idx], cur + ex, mask=win)
        return act & jnp.logical_not(win)

      lax.while_loop(cond, body, mask)
      return 0

    lax.fori_loop(0, NG, grp, 0)
    pltpu.sync_copy(ex_v, ex_h.at[k, wid])

  pltpu.sync_copy(den_v, shr.at[sid])
  plsc.subcore_barrier()
  base = pl.multiple_of(sid * SLICE, 8)
  pltpu.sync_copy(shr.at[0, pl.ds(base, SLICE)], acc_v)
  for t in range(1, NS):
    pltpu.sync_copy(shr.at[t, pl.ds(base, SLICE)], tmp_v)

    def ad(i, _):
      o = pl.multiple_of(i * 16, 16)
      acc_v[pl.ds(o, 16)] = acc_v[pl.ds(o, 16)] + tmp_v[pl.ds(o, 16)]
      return 0
    lax.fori_loop(0, SLICE // 16, ad, 0)
  pltpu.sync_copy(acc_v, den2_h.at[core, pl.ds(base, SLICE)])


def _kb(scores, m2, dstp):
  return pl.kernel(
      _kb_body,
      out_type=[
          jax.ShapeDtypeStruct((KH, NW, TP), jnp.float32),
          jax.ShapeDtypeStruct((NC, TAB), jnp.float32),
      ],
      mesh=_mesh(),
      scratch_types=[
          pltpu.VMEM((TAB,), jnp.float32),
          pltpu.VMEM((TAB,), jnp.float32),
          pltpu.VMEM((TP,), jnp.int32),
          pltpu.VMEM((TP,), jnp.float32),
          pltpu.VMEM((TP,), jnp.float32),
          pltpu.VMEM((64,), jnp.int32),
          pltpu.VMEM((2048,), jnp.float32),
          pltpu.VMEM((2048,), jnp.float32),
          pltpu.VMEM_SHARED((NS, TAB), jnp.float32),
          pltpu.VMEM((SLICE,), jnp.float32),
          pltpu.VMEM((SLICE,), jnp.float32),
      ],
  )(scores, m2, dstp)


def _kc_body(ex_h, den2_h, dst_h, src_h, hfeat_h, agg2_h,
             invd_v, dst_v, src_v, exc_v, attn_v, rows_v, zero_v,
             t0_v, t1_v, agg_s, sem):
  wid = _wid()
  sid = lax.axis_index("s")
  core = lax.axis_index("c")

  CKW = 2048
  for ci in range(TAB // CKW):
    pltpu.sync_copy(den2_h.at[0, pl.ds(ci * CKW, CKW)], t0_v)
    pltpu.sync_copy(den2_h.at[1, pl.ds(ci * CKW, CKW)], t1_v)

    def mrg(i, _, ci=ci):
      o = pl.multiple_of(i * 16, 16)
      dsum = t0_v[pl.ds(o, 16)] + t1_v[pl.ds(o, 16)]
      dsum = jnp.where(dsum == 0, jnp.float32(1.0), dsum)
      invd_v[pl.ds(pl.multiple_of(ci * CKW, 16) + o, 16)] = (
          jnp.float32(1.0) / dsum)
      return 0
    lax.fori_loop(0, CKW // 16, mrg, 0)

  zero_v[...] = jnp.zeros_like(zero_v)
  r0 = sid * ROWS_PER_TILE
  nfull = ROWS_PER_TILE // 16
  for rch in range(nfull):
    pltpu.sync_copy(zero_v, agg_s.at[pl.ds(r0 + rch * 16, 16), :])
  rem = ROWS_PER_TILE - nfull * 16
  if rem:
    pltpu.sync_copy(zero_v.at[pl.ds(0, rem)],
                    agg_s.at[pl.ds(r0 + nfull * 16, rem), :])
  plsc.subcore_barrier()

  pltpu.sync_copy(dst_h.at[wid], dst_v)
  pltpu.sync_copy(src_h.at[wid], src_v)

  def chunk(c, _):
    pltpu.async_copy(hfeat_h.at[src_v.at[c]], rows_v, sem).wait()
    for k in range(KH):
      pltpu.sync_copy(ex_h.at[k, wid, c], exc_v.at[k])
    for k in range(KH):
      for g in range(CH // 16):
        o = pl.multiple_of(g * 16, 16)
        d16 = dst_v[c, pl.ds(o, 16)]
        a16 = exc_v[k, pl.ds(o, 16)] * plsc.load_gather(
            invd_v, [d16 + k * NPH])
        attn_v[k, pl.ds(o, 16)] = a16

    def row(j, _):
      for k in range(KH):
        av = attn_v[k, j]
        for h in range(DOUT // 16):
          col = k * DOUT + h * 16
          rows_v[j, pl.ds(col, 16)] = rows_v[j, pl.ds(col, 16)] * av
      return 0
    lax.fori_loop(0, CH, row, 0)

    pltpu.sync_copy(rows_v, agg_s.at[dst_v.at[c]], add=True)
    return 0

  lax.fori_loop(0, TP // CH, chunk, 0)
  plsc.subcore_barrier()

  for rch in range(nfull):
    pltpu.sync_copy(agg_s.at[pl.ds(r0 + rch * 16, 16), :],
                    agg2_h.at[core, pl.ds(r0 + rch * 16, 16), :])
  if rem:
    pltpu.sync_copy(agg_s.at[pl.ds(r0 + nfull * 16, rem), :],
                    agg2_h.at[core, pl.ds(r0 + nfull * 16, rem), :])


def _kc(ex, den2, dstc, srcc, H):
  return pl.kernel(
      _kc_body,
      out_type=jax.ShapeDtypeStruct((NC, N, DH), jnp.float32),
      mesh=_mesh(),
      scratch_types=[
          pltpu.VMEM((TAB,), jnp.float32),
          pltpu.VMEM((TP // CH, CH), jnp.int32),
          pltpu.VMEM((TP // CH, CH), jnp.int32),
          pltpu.VMEM((KH, CH), jnp.float32),
          pltpu.VMEM((KH, CH), jnp.float32),
          pltpu.VMEM((CH, DH), jnp.float32),
          pltpu.VMEM((16, DH), jnp.float32),
          pltpu.VMEM((2048,), jnp.float32),
          pltpu.VMEM((2048,), jnp.float32),
          pltpu.VMEM_SHARED((N, DH), jnp.float32),
          pltpu.SemaphoreType.DMA,
      ],
  )(ex, den2, dstc, srcc, H)


def _tc2_body(a_ref, b_ref, o_ref):
  s = a_ref[...] + b_ref[...]
  o_ref[...] = jnp.where(s > 0, s, jnp.expm1(s))


def _tc2(aggA, aggB):
  blk = 1000
  return pl.pallas_call(
      _tc2_body,
      grid=(N // blk,),
      in_specs=[
          pl.BlockSpec((blk, DH), lambda i: (i, 0)),
          pl.BlockSpec((blk, DH), lambda i: (i, 0)),
      ],
      out_specs=pl.BlockSpec((blk, DH), lambda i: (i, 0)),
      out_shape=jax.ShapeDtypeStruct((N, DH), jnp.float32),
  )(aggA, aggB)


def kernel(X, edge_index, W, a):
  Wt = jnp.concatenate([W[k].T for k in range(KH)], axis=1)
  blocks = []
  for k in range(KH):
    col_s = jnp.zeros((DIN, 1), jnp.float32).at[
        k * DOUT:(k + 1) * DOUT, 0].set(a[k, 0, :DOUT])
    blocks.append(col_s)
  for k in range(KH):
    col_n = jnp.zeros((DIN, 1), jnp.float32).at[
        k * DOUT:(k + 1) * DOUT, 0].set(a[k, 0, DOUT:])
    blocks.append(col_n)
  Amat = jnp.concatenate(blocks, axis=1)

  dst = edge_index[0].astype(jnp.int32).reshape(NW, T)
  src = edge_index[1].astype(jnp.int32).reshape(NW, T)
  dstp = jnp.pad(dst, ((0, 0), (0, TP - T)))
  srcp = jnp.pad(src, ((0, 0), (0, TP - T)))

  H, SSt = _tc1(X, Wt, Amat)
  scores, m2 = _ka(SSt, dstp, srcp)
  ex, den2 = _kb(scores, m2, dstp)
  agg2 = _kc(ex, den2,
             dstp.reshape(NW, TP // CH, CH), srcp.reshape(NW, TP // CH, CH),
             H)
  return _tc2(agg2[0], agg2[1])


# speedup vs baseline: 30.9657x; 30.9657x over previous
import functools

import jax
import jax.numpy as jnp
from jax import lax
from jax.experimental import pallas as pl
from jax.experimental.pallas import tpu as pltpu
from jax.experimental.pallas import tpu_sc as plsc

N = 10000
E = 320000
DIN = 128
DOUT = 32
KH = 4
DH = KH * DOUT

NC = 2
NS = 16
NW = NC * NS
T = E // NW
CH = 128
TP = ((T + CH - 1) // CH) * CH
NG = TP // 16
NPH = 10240
TAB = KH * NPH
SLICE = TAB // NS
CKS = 4096
RPT = 624
HD = 2 * DOUT

_mesh = functools.partial(
    plsc.VectorSubcoreMesh, core_axis_name="c", subcore_axis_name="s",
    num_cores=NC, num_subcores=NS)


def _wid():
  return lax.axis_index("s") * NC + lax.axis_index("c")


def _lanes():
  return lax.broadcasted_iota(jnp.int32, (16,), 0)


def _tc1_body(x_ref, wt_ref, am_ref, h_ref, s_ref):
  x = x_ref[...]
  h = jnp.dot(x, wt_ref[...], preferred_element_type=jnp.float32)
  h_ref[...] = h
  s_ref[...] = jnp.dot(h, am_ref[...], preferred_element_type=jnp.float32)


def _tc1(X, Wt, Amat):
  blk = 1000
  grid = N // blk
  return pl.pallas_call(
      _tc1_body,
      grid=(grid,),
      in_specs=[
          pl.BlockSpec((blk, DIN), lambda i: (i, 0)),
          pl.BlockSpec((DIN, DH), lambda i: (0, 0)),
          pl.BlockSpec((DIN, 2 * KH), lambda i: (0, 0)),
      ],
      out_specs=[
          pl.BlockSpec((blk, DH), lambda i: (i, 0)),
          pl.BlockSpec((blk, 2 * KH), lambda i: (i, 0)),
      ],
      out_shape=[
          jax.ShapeDtypeStruct((N, DH), jnp.float32),
          jax.ShapeDtypeStruct((N, 2 * KH), jnp.float32),
      ],
  )(X, Wt, Amat)


def _ka_body(sst, dst_h, src_h, scores_h, m2_h,
             ss_v, sn_v, m_v, dst_v, src_v, sc_v, shr, acc_v, tmp_v):
  wid = _wid()
  sid = lax.axis_index("s")
  core = lax.axis_index("c")
  pltpu.sync_copy(dst_h.at[wid], dst_v)
  pltpu.sync_copy(src_h.at[wid], src_v)

  def init(i, _):
    m_v[pl.ds(pl.multiple_of(i * 16, 16), 16)] = jnp.full((16,), -1e30,
                                                          jnp.float32)
    return 0
  lax.fori_loop(0, TAB // 16, init, 0)

  lanes = _lanes()
  for k in range(KH):
    pltpu.sync_copy(sst.at[k], ss_v)
    pltpu.sync_copy(sst.at[KH + k], sn_v)

    def grp(g, _, k=k):
      off = pl.multiple_of(g * 16, 16)
      d = dst_v[pl.ds(off, 16)]
      s = src_v[pl.ds(off, 16)]
      mask = (g * 16 + lanes) < T
      sc = plsc.load_gather(ss_v, [d]) + plsc.load_gather(sn_v, [s])
      sc = jnp.where(sc > 0, sc, sc * jnp.float32(0.01))
      sc_v[pl.ds(off, 16)] = sc
      idx = d + k * NPH

      def cond(act):
        return jnp.any(act)

      def body(act):
        cur = plsc.load_gather(m_v, [idx])
        need = (sc > cur) & act
        plsc.store_scatter(m_v, [idx], sc, mask=need)
        cur2 = plsc.load_gather(m_v, [idx])
        return (sc > cur2) & act

      lax.while_loop(cond, body, mask)
      return 0

    lax.fori_loop(0, NG, grp, 0)
    pltpu.sync_copy(sc_v, scores_h.at[k, wid])

  for ci in range(TAB // CKS):
    pltpu.sync_copy(m_v.at[pl.ds(ci * CKS, CKS)], shr.at[sid])
    plsc.subcore_barrier()
    sub = pl.multiple_of(sid * (CKS // NS), 8)
    pltpu.sync_copy(shr.at[0, pl.ds(sub, CKS // NS)], acc_v)
    for t in range(1, NS):
      pltpu.sync_copy(shr.at[t, pl.ds(sub, CKS // NS)], tmp_v)

      def mx(i, _):
        o = pl.multiple_of(i * 16, 16)
        acc_v[pl.ds(o, 16)] = jnp.maximum(acc_v[pl.ds(o, 16)],
                                          tmp_v[pl.ds(o, 16)])
        return 0
      lax.fori_loop(0, (CKS // NS) // 16, mx, 0)
    pltpu.sync_copy(acc_v, m2_h.at[core, pl.ds(ci * CKS + sub, CKS // NS)])
    plsc.subcore_barrier()


def _ka(SSt, dstp, srcp):
  return pl.kernel(
      _ka_body,
      out_type=[
          jax.ShapeDtypeStruct((KH, NW, TP), jnp.float32),
          jax.ShapeDtypeStruct((NC, TAB), jnp.float32),
      ],
      mesh=_mesh(),
      compiler_params=pltpu.CompilerParams(needs_layout_passes=False, use_tc_tiling_on_sc=False),
      scratch_types=[
          pltpu.VMEM((N,), jnp.float32),
          pltpu.VMEM((N,), jnp.float32),
          pltpu.VMEM((TAB,), jnp.float32),
          pltpu.VMEM((TP,), jnp.int32),
          pltpu.VMEM((TP,), jnp.int32),
          pltpu.VMEM((TP,), jnp.float32),
          pltpu.VMEM_SHARED((NS, CKS), jnp.float32),
          pltpu.VMEM((CKS // NS,), jnp.float32),
          pltpu.VMEM((CKS // NS,), jnp.float32),
      ],
  )(SSt, dstp, srcp)


def _kb_body(scores_h, m2_h, dst_h, ex_h, den2_h,
             m_v, den_v, dst_v, sc_v, ex_v, claim_v, t0_v, t1_v, shr,
             acc_v, tmp_v):
  wid = _wid()
  sid = lax.axis_index("s")
  core = lax.axis_index("c")
  pltpu.sync_copy(dst_h.at[wid], dst_v)

  CKW = 2048
  for ci in range(TAB // CKW):
    pltpu.sync_copy(m2_h.at[0, pl.ds(ci * CKW, CKW)], t0_v)
    pltpu.sync_copy(m2_h.at[1, pl.ds(ci * CKW, CKW)], t1_v)

    def mrg(i, _, ci=ci):
      o = pl.multiple_of(i * 16, 16)
      m_v[pl.ds(pl.multiple_of(ci * CKW, 16) + o, 16)] = jnp.maximum(
          t0_v[pl.ds(o, 16)], t1_v[pl.ds(o, 16)])
      return 0
    lax.fori_loop(0, CKW // 16, mrg, 0)

  def zero(i, _):
    den_v[pl.ds(pl.multiple_of(i * 16, 16), 16)] = jnp.zeros((16,),
                                                             jnp.float32)
    return 0
  lax.fori_loop(0, TAB // 16, zero, 0)

  lanes = _lanes()
  for k in range(KH):
    pltpu.sync_copy(scores_h.at[k, wid], sc_v)

    def grp(g, _, k=k):
      off = pl.multiple_of(g * 16, 16)
      d = dst_v[pl.ds(off, 16)]
      sc = sc_v[pl.ds(off, 16)]
      mask = (g * 16 + lanes) < T
      idx = d + k * NPH
      mx = plsc.load_gather(m_v, [idx])
      ex = jnp.where(mask, jnp.exp(sc - mx), jnp.float32(0.0))
      ex_v[pl.ds(off, 16)] = ex
      slot = lax.bitwise_and(idx, jnp.int32(63))

      def cond(act):
        return jnp.any(act)

      def body(act):
        plsc.store_scatter(claim_v, [slot], lanes, mask=act)
        win = (plsc.load_gather(claim_v, [slot]) == lanes) & act
        cur = plsc.load_gather(den_v, [idx])
        plsc.store_scatter(den_v, [idx], cur + ex, mask=win)
        return act & jnp.logical_not(win)

      lax.while_loop(cond, body, mask)
      return 0

    lax.fori_loop(0, NG, grp, 0)
    pltpu.sync_copy(ex_v, ex_h.at[k, wid])

  for ci in range(TAB // CKS):
    pltpu.sync_copy(den_v.at[pl.ds(ci * CKS, CKS)], shr.at[sid])
    plsc.subcore_barrier()
    sub = pl.multiple_of(sid * (CKS // NS), 8)
    pltpu.sync_copy(shr.at[0, pl.ds(sub, CKS // NS)], acc_v)
    for t in range(1, NS):
      pltpu.sync_copy(shr.at[t, pl.ds(sub, CKS // NS)], tmp_v)

      def ad(i, _):
        o = pl.multiple_of(i * 16, 16)
        acc_v[pl.ds(o, 16)] = acc_v[pl.ds(o, 16)] + tmp_v[pl.ds(o, 16)]
        return 0
      lax.fori_loop(0, (CKS // NS) // 16, ad, 0)
    pltpu.sync_copy(acc_v, den2_h.at[core, pl.ds(ci * CKS + sub, CKS // NS)])
    plsc.subcore_barrier()


def _kb(scores, m2, dstp):
  return pl.kernel(
      _kb_body,
      out_type=[
          jax.ShapeDtypeStruct((KH, NW, TP), jnp.float32),
          jax.ShapeDtypeStruct((NC, TAB), jnp.float32),
      ],
      mesh=_mesh(),
      compiler_params=pltpu.CompilerParams(needs_layout_passes=False, use_tc_tiling_on_sc=False),
      scratch_types=[
          pltpu.VMEM((TAB,), jnp.float32),
          pltpu.VMEM((TAB,), jnp.float32),
          pltpu.VMEM((TP,), jnp.int32),
          pltpu.VMEM((TP,), jnp.float32),
          pltpu.VMEM((TP,), jnp.float32),
          pltpu.VMEM((64,), jnp.int32),
          pltpu.VMEM((2048,), jnp.float32),
          pltpu.VMEM((2048,), jnp.float32),
          pltpu.VMEM_SHARED((NS, CKS), jnp.float32),
          pltpu.VMEM((CKS // NS,), jnp.float32),
          pltpu.VMEM((CKS // NS,), jnp.float32),
      ],
  )(scores, m2, dstp)


def _kc_body(ex_h, den2_h, dst_h, src_h, h0_h, h1_h, agg2_h,
             invd_v, dst_v, src_v, exc_v, rows_v, zero_v,
             t0_v, t1_v, agg_s, sem):
  wid = _wid()
  sid = lax.axis_index("s")
  core = lax.axis_index("c")

  CKW = 2048
  for ci in range(TAB // CKW):
    pltpu.sync_copy(den2_h.at[0, pl.ds(ci * CKW, CKW)], t0_v)
    pltpu.sync_copy(den2_h.at[1, pl.ds(ci * CKW, CKW)], t1_v)

    def mrg(i, _, ci=ci):
      o = pl.multiple_of(i * 16, 16)
      dsum = t0_v[pl.ds(o, 16)] + t1_v[pl.ds(o, 16)]
      dsum = jnp.where(dsum == 0, jnp.float32(1.0), dsum)
      invd_v[pl.ds(pl.multiple_of(ci * CKW, 16) + o, 16)] = (
          jnp.float32(1.0) / dsum)
      return 0
    lax.fori_loop(0, CKW // 16, mrg, 0)

  pltpu.sync_copy(dst_h.at[wid], dst_v)
  pltpu.sync_copy(src_h.at[wid], src_v)

  zr16 = jnp.zeros((16,), jnp.float32)
  for r in range(16):
    for h in range(HD // 16):
      zero_v[r, pl.ds(h * 16, 16)] = zr16
  r0 = sid * RPT

  for p in range(2):
    h_h = h0_h if p == 0 else h1_h
    for rch in range(RPT // 16):
      pltpu.sync_copy(zero_v,
                      agg_s.at[pl.ds(pl.multiple_of(r0 + rch * 16, 8), 16), :])

    @pl.when(sid == NS - 1)
    def _():
      pltpu.sync_copy(zero_v, agg_s.at[pl.ds(N - 16, 16), :])
    plsc.subcore_barrier()

    def chunk(c, _, p=p, h_h=h_h):
      pltpu.async_copy(h_h.at[src_v.at[c]], rows_v, sem).wait()
      for k in range(2):
        pltpu.sync_copy(ex_h.at[2 * p + k, wid, c], exc_v.at[k])

      def grp16(g, _):
        o = pl.multiple_of(g * 16, 16)
        d16 = dst_v[c, pl.ds(o, 16)]
        a16 = []
        for k in range(2):
          a16.append(exc_v[k, pl.ds(o, 16)] * plsc.load_gather(
              invd_v, [d16 + (2 * p + k) * NPH]))
        for jj in range(16):
          j = g * 16 + jj
          for k in range(2):
            av = a16[k][jj]
            for h in range(DOUT // 16):
              col = k * DOUT + h * 16
              rows_v[j, pl.ds(col, 16)] = rows_v[j, pl.ds(col, 16)] * av
        return 0
      lax.fori_loop(0, CH // 16, grp16, 0)

      pltpu.sync_copy(rows_v, agg_s.at[dst_v.at[c]], add=True)
      return 0

    lax.fori_loop(0, TP // CH, chunk, 0)
    plsc.subcore_barrier()

    for rch in range(RPT // 16):
      ro = pl.multiple_of(r0 + rch * 16, 8)
      pltpu.sync_copy(agg_s.at[pl.ds(ro, 16), :],
                      agg2_h.at[core, p, pl.ds(ro, 16), :])

    @pl.when(sid == NS - 1)
    def _():
      pltpu.sync_copy(agg_s.at[pl.ds(N - 16, 16), :],
                      agg2_h.at[core, p, pl.ds(N - 16, 16), :])
    plsc.subcore_barrier()


def _kc(ex, den2, dstc, srcc, H0, H1):
  return pl.kernel(
      _kc_body,
      out_type=jax.ShapeDtypeStruct((NC, 2, N, HD), jnp.float32),
      mesh=_mesh(),
      compiler_params=pltpu.CompilerParams(needs_layout_passes=False, use_tc_tiling_on_sc=False),
      scratch_types=[
          pltpu.VMEM((TAB,), jnp.float32),
          pltpu.VMEM((TP // CH, CH), jnp.int32),
          pltpu.VMEM((TP // CH, CH), jnp.int32),
          pltpu.VMEM((2, CH), jnp.float32),
          pltpu.VMEM((CH, HD), jnp.float32),
          pltpu.VMEM((16, HD), jnp.float32),
          pltpu.VMEM((2048,), jnp.float32),
          pltpu.VMEM((2048,), jnp.float32),
          pltpu.VMEM_SHARED((N, HD), jnp.float32),
          pltpu.SemaphoreType.DMA,
      ],
  )(ex, den2, dstc, srcc, H0, H1)


def _tc2_body(a0_ref, a1_ref, b0_ref, b1_ref, o_ref):
  s0 = a0_ref[...] + b0_ref[...]
  s1 = a1_ref[...] + b1_ref[...]
  o_ref[:, :HD] = jnp.where(s0 > 0, s0, jnp.exp(s0) - jnp.float32(1.0))
  o_ref[:, HD:] = jnp.where(s1 > 0, s1, jnp.exp(s1) - jnp.float32(1.0))


def _tc2(a0, a1, b0, b1):
  blk = 1000
  half = pl.BlockSpec((blk, HD), lambda i: (i, 0))
  return pl.pallas_call(
      _tc2_body,
      grid=(N // blk,),
      in_specs=[half, half, half, half],
      out_specs=pl.BlockSpec((blk, DH), lambda i: (i, 0)),
      out_shape=jax.ShapeDtypeStruct((N, DH), jnp.float32),
  )(a0, a1, b0, b1)


def kernel(X, edge_index, W, a):
  Wt = jnp.concatenate([W[k].T for k in range(KH)], axis=1)
  blocks = []
  for k in range(KH):
    col_s = jnp.zeros((DIN, 1), jnp.float32).at[
        k * DOUT:(k + 1) * DOUT, 0].set(a[k, 0, :DOUT])
    blocks.append(col_s)
  for k in range(KH):
    col_n = jnp.zeros((DIN, 1), jnp.float32).at[
        k * DOUT:(k + 1) * DOUT, 0].set(a[k, 0, DOUT:])
    blocks.append(col_n)
  Amat = jnp.concatenate(blocks, axis=1)

  dst = edge_index[0].astype(jnp.int32).reshape(NW, T)
  src = edge_index[1].astype(jnp.int32).reshape(NW, T)
  dstp = jnp.pad(dst, ((0, 0), (0, TP - T)))
  srcp = jnp.pad(src, ((0, 0), (0, TP - T)))

  H, S = _tc1(X, Wt, Amat)
  SSt = S.T
  scores, m2 = _ka(SSt, dstp, srcp)
  ex, den2 = _kb(scores, m2, dstp)
  agg2 = _kc(ex.reshape(KH, NW, TP // CH, CH), den2,
             dstp.reshape(NW, TP // CH, CH), srcp.reshape(NW, TP // CH, CH),
             H[:, :HD], H[:, HD:])
  return _tc2(agg2[0, 0], agg2[0, 1], agg2[1, 0], agg2[1, 1])
